# Initial kernel scaffold; baseline (speedup 1.0000x reference)
import jax
import jax.numpy as jnp
from jax.experimental import pallas as pl


def kernel(x, edge_index, W1, as1, ad1, b1, W2, as2, ad2, b2, W3, as3, ad3, b3, Wout, bout):
    raise NotImplementedError("write your pallas kernel here")


# speedup vs baseline: 64.5302x; 64.5302x over previous
import functools

import jax
import jax.numpy as jnp
from jax import lax
from jax.experimental import pallas as pl
from jax.experimental.pallas import tpu as pltpu
from jax.experimental.pallas import tpu_sc as plsc

N_NODES = 50000
N_PAD = 50048
N_EDGES = 1600000
BLK = 2000
GRID = N_NODES // BLK
J = 128
TOT_STEPS = N_EDGES // J
NC, NS = 2, 16
NW = NC * NS


def _leaky(v):
    return jnp.maximum(v, 0.2 * v)


def _elu(v):
    return jnp.where(v > 0, v, jnp.exp(jnp.minimum(v, 0.0)) - 1.0)


def _t1_body(x_ref, w_ref, asm_ref, adm_ref, tab_ref, adst_ref, m_ref):
    i = pl.program_id(0)
    h = jnp.dot(x_ref[...], w_ref[...], preferred_element_type=jnp.float32)
    asrc = jnp.dot(h, asm_ref[...], preferred_element_type=jnp.float32)
    adst = jnp.dot(h, adm_ref[...], preferred_element_type=jnp.float32)
    nblk = h.shape[0]
    tab_ref[...] = jnp.concatenate(
        [h, asrc, jnp.zeros((nblk, 14), jnp.float32)], axis=1)
    adst_ref[...] = jnp.concatenate(
        [adst, jnp.zeros((nblk, 14), jnp.float32)], axis=1)

    @pl.when(i == 0)
    def _():
        m_ref[...] = jnp.full(m_ref.shape, -jnp.inf, jnp.float32)

    m_ref[...] = jnp.maximum(m_ref[...], jnp.max(asrc, axis=0, keepdims=True))


def _t1(x, w1, asm, adm):
    return pl.pallas_call(
        _t1_body,
        grid=(GRID,),
        in_specs=[
            pl.BlockSpec((BLK, 3), lambda i: (i, 0)),
            pl.BlockSpec((3, 32), lambda i: (0, 0)),
            pl.BlockSpec((32, 2), lambda i: (0, 0)),
            pl.BlockSpec((32, 2), lambda i: (0, 0)),
        ],
        out_specs=[
            pl.BlockSpec((BLK, 48), lambda i: (i, 0)),
            pl.BlockSpec((BLK, 16), lambda i: (i, 0)),
            pl.BlockSpec((1, 2), lambda i: (0, 0)),
        ],
        out_shape=[
            jax.ShapeDtypeStruct((N_NODES, 48), jnp.float32),
            jax.ShapeDtypeStruct((N_NODES, 16), jnp.float32),
            jax.ShapeDtypeStruct((1, 2), jnp.float32),
        ],
    )(x, w1, asm, adm)


def _make_merge(hout, cout):
    hcout = hout * cout
    tw_tab = 48 if hout == 2 else 16

    def body(acc_ref, tab_ref, adst_ref, m_ref, b_ref, w_ref, asm_ref,
             adm_ref, tabo_ref, adsto_ref, mo_ref):
        i = pl.program_id(0)
        adst = adst_ref[:, 0:2]
        ub = _leaky(m_ref[...] + adst)
        asrc = tab_ref[:, 32:34]
        p_self = jnp.exp(_leaky(asrc + adst) - ub)
        den = acc_ref[0, :, 16:18] + p_self
        hmat = tab_ref[:, 0:32]
        nblk = hmat.shape[0]
        pexp = jnp.concatenate(
            [jnp.broadcast_to(p_self[:, k:k + 1], (nblk, 16)) for k in range(2)],
            axis=1)
        dexp = jnp.concatenate(
            [jnp.broadcast_to(den[:, k:k + 1], (nblk, 16)) for k in range(2)],
            axis=1)
        msg = jnp.concatenate(
            [acc_ref[0, :, 0:16], acc_ref[1, :, 0:16]], axis=1) + hmat * pexp
        y = msg / (dexp + 1e-16) + b_ref[...]
        xn = _elu(y)
        h2 = jnp.dot(xn, w_ref[...], preferred_element_type=jnp.float32)
        asrc2 = jnp.dot(h2, asm_ref[...], preferred_element_type=jnp.float32)
        adst2 = jnp.dot(h2, adm_ref[...], preferred_element_type=jnp.float32)
        if hout == 2:
            tabo_ref[...] = jnp.concatenate(
                [h2, asrc2, jnp.zeros((nblk, 14), jnp.float32)], axis=1)
            adsto_ref[...] = jnp.concatenate(
                [adst2, jnp.zeros((nblk, 14), jnp.float32)], axis=1)
        else:
            tabo_ref[...] = jnp.concatenate(
                [h2, asrc2, jnp.zeros((nblk, 7), jnp.float32)], axis=1)
            adsto_ref[...] = jnp.concatenate(
                [jnp.zeros((nblk, 8), jnp.float32), adst2,
                 jnp.zeros((nblk, 7), jnp.float32)], axis=1)

        @pl.when(i == 0)
        def _():
            mo_ref[...] = jnp.full(mo_ref.shape, -jnp.inf, jnp.float32)

        mo_ref[...] = jnp.maximum(mo_ref[...],
                                  jnp.max(asrc2, axis=0, keepdims=True))

    def run(acc, tab, adst, m, b, w, asm, adm):
        return pl.pallas_call(
            body,
            grid=(GRID,),
            in_specs=[
                pl.BlockSpec((2, BLK, 32), lambda i: (0, i, 0)),
                pl.BlockSpec((BLK, 48), lambda i: (i, 0)),
                pl.BlockSpec((BLK, 16), lambda i: (i, 0)),
                pl.BlockSpec((1, 2), lambda i: (0, 0)),
                pl.BlockSpec((1, 32), lambda i: (0, 0)),
                pl.BlockSpec((32, hcout), lambda i: (0, 0)),
                pl.BlockSpec((hcout, hout), lambda i: (0, 0)),
                pl.BlockSpec((hcout, hout), lambda i: (0, 0)),
            ],
            out_specs=[
                pl.BlockSpec((BLK, tw_tab), lambda i: (i, 0)),
                pl.BlockSpec((BLK, 16), lambda i: (i, 0)),
                pl.BlockSpec((1, hout), lambda i: (0, 0)),
            ],
            out_shape=[
                jax.ShapeDtypeStruct((N_NODES, tw_tab), jnp.float32),
                jax.ShapeDtypeStruct((N_NODES, 16), jnp.float32),
                jax.ShapeDtypeStruct((1, hout), jnp.float32),
            ],
        )(acc, tab, adst, m, b, w, asm, adm)

    return run


def _t4_body(acc_ref, tab_ref, adst_ref, m_ref, b_ref, w_ref, bo_ref, out_ref):
    adst = adst_ref[:, 8:9]
    ub = _leaky(m_ref[...] + adst)
    asrc = tab_ref[:, 8:9]
    p_self = jnp.exp(_leaky(asrc + adst) - ub)
    den = acc_ref[0, :, 8:9] + acc_ref[1, :, 8:9] + p_self
    msg = acc_ref[0, :, 16:24] + acc_ref[1, :, 16:24] + tab_ref[:, 0:8] * p_self
    y = msg / (den + 1e-16) + b_ref[...]
    xn = _elu(y)
    out_ref[...] = jnp.dot(xn, w_ref[...],
                           preferred_element_type=jnp.float32) + bo_ref[...]


def _t4(acc, tab, adst, m, b3, wout, bout):
    return pl.pallas_call(
        _t4_body,
        grid=(GRID,),
        in_specs=[
            pl.BlockSpec((2, BLK, 32), lambda i: (0, i, 0)),
            pl.BlockSpec((BLK, 16), lambda i: (i, 0)),
            pl.BlockSpec((BLK, 16), lambda i: (i, 0)),
            pl.BlockSpec((1, 1), lambda i: (0, 0)),
            pl.BlockSpec((1, 8), lambda i: (0, 0)),
            pl.BlockSpec((8, 1), lambda i: (0, 0)),
            pl.BlockSpec((1, 1), lambda i: (0, 0)),
        ],
        out_specs=pl.BlockSpec((BLK, 1), lambda i: (i, 0)),
        out_shape=jax.ShapeDtypeStruct((N_NODES, 1), jnp.float32),
    )(acc, tab, adst, m, b3, wout, bout)


def _make_edge2_kernel():
    rpt = N_PAD // NS
    mesh = plsc.VectorSubcoreMesh(core_axis_name="c", subcore_axis_name="s",
                                  num_cores=NC, num_subcores=NS)

    @functools.partial(
        pl.kernel,
        mesh=mesh,
        compiler_params=pltpu.CompilerParams(use_tc_tiling_on_sc=False),
        out_type=jax.ShapeDtypeStruct((NC, N_PAD, 32), jnp.float32),
        scratch_types=[
            pltpu.VMEM((J,), jnp.int32),
            pltpu.VMEM((J,), jnp.int32),
            pltpu.VMEM((J, 48), jnp.float32),
            pltpu.VMEM((J, 16), jnp.float32),
            pltpu.VMEM((J, 32), jnp.float32),
            pltpu.VMEM((16,), jnp.float32),
            pltpu.VMEM_SHARED((N_PAD, 32), jnp.float32),
        ],
    )
    def k(src_hbm, dst_hbm, tab_hbm, adst_hbm, zeros_hbm, consts_hbm,
          out_hbm, src_v, dst_v, rows_v, ad_v, msg_v, consts_v, acc_sh):
        cid = lax.axis_index("c")
        sid = lax.axis_index("s")

        pltpu.sync_copy(consts_hbm, consts_v)
        pltpu.sync_copy(zeros_hbm.at[pl.ds(sid * rpt, rpt)],
                        acc_sh.at[pl.ds(sid * rpt, rpt)])
        plsc.subcore_barrier()

        cv = consts_v[...]
        nsteps = (TOT_STEPS - sid + NS - 1) // NS

        def body(t, carry):
            base = (sid + t * NS) * J
            pltpu.sync_copy(src_hbm.at[pl.ds(base, J)], src_v)
            pltpu.sync_copy(dst_hbm.at[pl.ds(base, J)], dst_v)
            pltpu.sync_copy(tab_hbm.at[src_v], rows_v)
            pltpu.sync_copy(adst_hbm.at[dst_v], ad_v)
            for j in range(J):
                la = ad_v[j, pl.ds(0, 16)]
                sa = rows_v[j, pl.ds(32, 16)]
                pvec = jnp.exp(_leaky(sa + la) - _leaky(cv + la))
                h0 = rows_v[j, pl.ds(0, 16)]
                h1 = rows_v[j, pl.ds(16, 16)]
                hsel = jnp.where(cid == 0, h0, h1)
                psel = jnp.where(cid == 0,
                                 jnp.broadcast_to(pvec[0], (16,)),
                                 jnp.broadcast_to(pvec[1], (16,)))
                msg_v[j, pl.ds(0, 16)] = hsel * psel
                msg_v[j, pl.ds(16, 16)] = pvec
            pltpu.sync_copy(msg_v, acc_sh.at[dst_v], add=True)
            return carry

        lax.fori_loop(0, nsteps, body, 0)
        plsc.subcore_barrier()
        pltpu.sync_copy(acc_sh.at[pl.ds(sid * rpt, rpt)],
                        out_hbm.at[cid, pl.ds(sid * rpt, rpt)])

    return k


def _make_edge1_kernel():
    rpt = N_PAD // NS
    mesh = plsc.VectorSubcoreMesh(core_axis_name="c", subcore_axis_name="s",
                                  num_cores=NC, num_subcores=NS)

    @functools.partial(
        pl.kernel,
        mesh=mesh,
        compiler_params=pltpu.CompilerParams(use_tc_tiling_on_sc=False),
        out_type=jax.ShapeDtypeStruct((NC, N_PAD, 32), jnp.float32),
        scratch_types=[
            pltpu.VMEM((J,), jnp.int32),
            pltpu.VMEM((J,), jnp.int32),
            pltpu.VMEM((J, 16), jnp.float32),
            pltpu.VMEM((J, 16), jnp.float32),
            pltpu.VMEM((J, 32), jnp.float32),
            pltpu.VMEM((16,), jnp.float32),
            pltpu.VMEM_SHARED((N_PAD, 32), jnp.float32),
        ],
    )
    def k(src_hbm, dst_hbm, tab_hbm, adst_hbm, zeros_hbm, consts_hbm,
          out_hbm, src_v, dst_v, rows_v, ad_v, msg_v, consts_v, acc_sh):
        cid = lax.axis_index("c")
        sid = lax.axis_index("s")
        wid = sid * NC + cid

        pltpu.sync_copy(consts_hbm, consts_v)
        pltpu.sync_copy(zeros_hbm.at[pl.ds(sid * rpt, rpt)],
                        acc_sh.at[pl.ds(sid * rpt, rpt)])
        plsc.subcore_barrier()

        cv = consts_v[...]
        nsteps = (TOT_STEPS - wid + NW - 1) // NW

        def body(t, carry):
            base = (wid + t * NW) * J
            pltpu.sync_copy(src_hbm.at[pl.ds(base, J)], src_v)
            pltpu.sync_copy(dst_hbm.at[pl.ds(base, J)], dst_v)
            pltpu.sync_copy(tab_hbm.at[src_v], rows_v)
            pltpu.sync_copy(adst_hbm.at[dst_v], ad_v)
            for j in range(J):
                la = ad_v[j, pl.ds(0, 16)]
                rv = rows_v[j, pl.ds(0, 16)]
                pvec = jnp.exp(_leaky(rv + la) - _leaky(cv + la))
                p0 = jnp.broadcast_to(pvec[8], (16,))
                msg_v[j, pl.ds(0, 16)] = pvec
                msg_v[j, pl.ds(16, 16)] = rv * p0
            pltpu.sync_copy(msg_v, acc_sh.at[dst_v], add=True)
            return carry

        lax.fori_loop(0, nsteps, body, 0)
        plsc.subcore_barrier()
        pltpu.sync_copy(acc_sh.at[pl.ds(sid * rpt, rpt)],
                        out_hbm.at[cid, pl.ds(sid * rpt, rpt)])

    return k


@functools.lru_cache(maxsize=None)
def _edge_kernel(heads):
    return _make_edge2_kernel() if heads == 2 else _make_edge1_kernel()


def _att_mat(att):
    heads, chan = att.shape
    cols = []
    for h in range(heads):
        parts = [jnp.zeros((chan,), jnp.float32)] * heads
        parts[h] = att[h]
        cols.append(jnp.concatenate(parts))
    return jnp.stack(cols, axis=1)


def kernel(x, edge_index, W1, as1, ad1, b1, W2, as2, ad2, b2, W3, as3, ad3,
           b3, Wout, bout):
    ei = edge_index.astype(jnp.int32)
    src, dst = ei[0], ei[1]
    zeros32 = jnp.zeros((N_PAD, 32), jnp.float32)

    asm1, adm1 = _att_mat(as1), _att_mat(ad1)
    asm2, adm2 = _att_mat(as2), _att_mat(ad2)
    asm3, adm3 = _att_mat(as3), _att_mat(ad3)

    tab1, adst1, m1 = _t1(x, W1, asm1, adm1)
    c1 = jnp.pad(m1[0], (0, 14))
    acc1 = _edge_kernel(2)(src, dst, tab1, adst1, zeros32, c1)

    t2 = _make_merge(2, 16)
    tab2, adst2, m2 = t2(acc1, tab1, adst1, m1, b1[None, :], W2, asm2, adm2)
    c2 = jnp.pad(m2[0], (0, 14))
    acc2 = _edge_kernel(2)(src, dst, tab2, adst2, zeros32, c2)

    t3 = _make_merge(1, 8)
    tab3, adst3, m3 = t3(acc2, tab2, adst2, m2, b2[None, :], W3, asm3, adm3)
    c3 = jnp.pad(m3[0], (8, 7))
    acc3 = _edge_kernel(1)(src, dst, tab3, adst3, zeros32, c3)

    return _t4(acc3, tab3, adst3, m3, b3[None, :], Wout, bout[None, :])


# speedup vs baseline: 185.3299x; 2.8720x over previous
import functools

import jax
import jax.numpy as jnp
from jax import lax
from jax.experimental import pallas as pl
from jax.experimental.pallas import tpu as pltpu
from jax.experimental.pallas import tpu_sc as plsc

N_NODES = 50000
N_PAD = 50048
N_EDGES = 1600000
BLK = 5000
GRID = N_NODES // BLK
J = 128
TOT_STEPS = N_EDGES // J
NC, NS = 2, 16
NW = NC * NS


def _leaky(v):
    return jnp.maximum(v, 0.2 * v)


def _elu(v):
    return jnp.where(v > 0, v, jnp.exp(jnp.minimum(v, 0.0)) - 1.0)


def _t1_body(x_ref, w_ref, asm_ref, adm_ref, tab_ref, adst_ref, m_ref):
    i = pl.program_id(0)
    h = jnp.dot(x_ref[...], w_ref[...], preferred_element_type=jnp.float32)
    asrc = jnp.dot(h, asm_ref[...], preferred_element_type=jnp.float32)
    adst = jnp.dot(h, adm_ref[...], preferred_element_type=jnp.float32)
    nblk = h.shape[0]
    tab_ref[...] = jnp.concatenate(
        [h, asrc, jnp.zeros((nblk, 14), jnp.float32)], axis=1)
    adst_ref[...] = jnp.concatenate(
        [adst, jnp.zeros((nblk, 14), jnp.float32)], axis=1)

    @pl.when(i == 0)
    def _():
        m_ref[...] = jnp.full(m_ref.shape, -jnp.inf, jnp.float32)

    m_ref[...] = jnp.maximum(m_ref[...], jnp.max(asrc, axis=0, keepdims=True))


def _t1(x, w1, asm, adm):
    return pl.pallas_call(
        _t1_body,
        grid=(GRID,),
        in_specs=[
            pl.BlockSpec((BLK, 3), lambda i: (i, 0)),
            pl.BlockSpec((3, 32), lambda i: (0, 0)),
            pl.BlockSpec((32, 2), lambda i: (0, 0)),
            pl.BlockSpec((32, 2), lambda i: (0, 0)),
        ],
        out_specs=[
            pl.BlockSpec((BLK, 48), lambda i: (i, 0)),
            pl.BlockSpec((BLK, 16), lambda i: (i, 0)),
            pl.BlockSpec((1, 2), lambda i: (0, 0)),
        ],
        out_shape=[
            jax.ShapeDtypeStruct((N_NODES, 48), jnp.float32),
            jax.ShapeDtypeStruct((N_NODES, 16), jnp.float32),
            jax.ShapeDtypeStruct((1, 2), jnp.float32),
        ],
    )(x, w1, asm, adm)


def _make_merge(hout, cout):
    hcout = hout * cout
    tw_tab = 48 if hout == 2 else 16

    def body(acc_ref, tab_ref, adst_ref, m_ref, b_ref, w_ref, asm_ref,
             adm_ref, tabo_ref, adsto_ref, mo_ref):
        i = pl.program_id(0)
        adst = adst_ref[:, 0:2]
        ub = _leaky(m_ref[...] + adst)
        asrc = tab_ref[:, 32:34]
        p_self = jnp.exp(_leaky(asrc + adst) - ub)
        den = jnp.concatenate(
            [acc_ref[0, :, 16:17], acc_ref[1, :, 16:17]], axis=1) + p_self
        hmat = tab_ref[:, 0:32]
        nblk = hmat.shape[0]
        pexp = jnp.concatenate(
            [jnp.broadcast_to(p_self[:, k:k + 1], (nblk, 16)) for k in range(2)],
            axis=1)
        dexp = jnp.concatenate(
            [jnp.broadcast_to(den[:, k:k + 1], (nblk, 16)) for k in range(2)],
            axis=1)
        msg = jnp.concatenate(
            [acc_ref[0, :, 0:16], acc_ref[1, :, 0:16]], axis=1) + hmat * pexp
        y = msg / (dexp + 1e-16) + b_ref[...]
        xn = _elu(y)
        h2 = jnp.dot(xn, w_ref[...], preferred_element_type=jnp.float32)
        asrc2 = jnp.dot(h2, asm_ref[...], preferred_element_type=jnp.float32)
        adst2 = jnp.dot(h2, adm_ref[...], preferred_element_type=jnp.float32)
        if hout == 2:
            tabo_ref[...] = jnp.concatenate(
                [h2, asrc2, jnp.zeros((nblk, 14), jnp.float32)], axis=1)
            adsto_ref[...] = jnp.concatenate(
                [adst2, jnp.zeros((nblk, 14), jnp.float32)], axis=1)
        else:
            tabo_ref[...] = jnp.concatenate(
                [h2, asrc2, jnp.zeros((nblk, 7), jnp.float32)], axis=1)
            adsto_ref[...] = jnp.concatenate(
                [jnp.zeros((nblk, 8), jnp.float32), adst2,
                 jnp.zeros((nblk, 7), jnp.float32)], axis=1)

        @pl.when(i == 0)
        def _():
            mo_ref[...] = jnp.full(mo_ref.shape, -jnp.inf, jnp.float32)

        mo_ref[...] = jnp.maximum(mo_ref[...],
                                  jnp.max(asrc2, axis=0, keepdims=True))

    def run(acc, tab, adst, m, b, w, asm, adm):
        return pl.pallas_call(
            body,
            grid=(GRID,),
            in_specs=[
                pl.BlockSpec((2, BLK, 32), lambda i: (0, i, 0)),
                pl.BlockSpec((BLK, 48), lambda i: (i, 0)),
                pl.BlockSpec((BLK, 16), lambda i: (i, 0)),
                pl.BlockSpec((1, 2), lambda i: (0, 0)),
                pl.BlockSpec((1, 32), lambda i: (0, 0)),
                pl.BlockSpec((32, hcout), lambda i: (0, 0)),
                pl.BlockSpec((hcout, hout), lambda i: (0, 0)),
                pl.BlockSpec((hcout, hout), lambda i: (0, 0)),
            ],
            out_specs=[
                pl.BlockSpec((BLK, tw_tab), lambda i: (i, 0)),
                pl.BlockSpec((BLK, 16), lambda i: (i, 0)),
                pl.BlockSpec((1, hout), lambda i: (0, 0)),
            ],
            out_shape=[
                jax.ShapeDtypeStruct((N_NODES, tw_tab), jnp.float32),
                jax.ShapeDtypeStruct((N_NODES, 16), jnp.float32),
                jax.ShapeDtypeStruct((1, hout), jnp.float32),
            ],
        )(acc, tab, adst, m, b, w, asm, adm)

    return run


def _t4_body(acc_ref, tab_ref, adst_ref, m_ref, b_ref, w_ref, bo_ref, out_ref):
    adst = adst_ref[:, 8:9]
    ub = _leaky(m_ref[...] + adst)
    asrc = tab_ref[:, 8:9]
    p_self = jnp.exp(_leaky(asrc + adst) - ub)
    den = acc_ref[0, :, 16:17] + acc_ref[1, :, 16:17] + p_self
    msg = acc_ref[0, :, 0:8] + acc_ref[1, :, 0:8] + tab_ref[:, 0:8] * p_self
    y = msg / (den + 1e-16) + b_ref[...]
    xn = _elu(y)
    out_ref[...] = jnp.dot(xn, w_ref[...],
                           preferred_element_type=jnp.float32) + bo_ref[...]


def _t4(acc, tab, adst, m, b3, wout, bout):
    return pl.pallas_call(
        _t4_body,
        grid=(GRID,),
        in_specs=[
            pl.BlockSpec((2, BLK, 32), lambda i: (0, i, 0)),
            pl.BlockSpec((BLK, 16), lambda i: (i, 0)),
            pl.BlockSpec((BLK, 16), lambda i: (i, 0)),
            pl.BlockSpec((1, 1), lambda i: (0, 0)),
            pl.BlockSpec((1, 8), lambda i: (0, 0)),
            pl.BlockSpec((8, 1), lambda i: (0, 0)),
            pl.BlockSpec((1, 1), lambda i: (0, 0)),
        ],
        out_specs=pl.BlockSpec((BLK, 1), lambda i: (i, 0)),
        out_shape=jax.ShapeDtypeStruct((N_NODES, 1), jnp.float32),
    )(acc, tab, adst, m, b3, wout, bout)


G = J // 16
TRIP2 = 782
TRIP1 = 392
EP = 1613824


def _make_edge2_kernel():
    rpt = N_PAD // NS
    mesh = plsc.VectorSubcoreMesh(core_axis_name="c", subcore_axis_name="s",
                                  num_cores=NC, num_subcores=NS)

    @functools.partial(
        pl.kernel,
        mesh=mesh,
        compiler_params=pltpu.CompilerParams(use_tc_tiling_on_sc=False),
        out_type=jax.ShapeDtypeStruct((NC, N_PAD, 32), jnp.float32),
        scratch_types=[
            pltpu.VMEM((2, J), jnp.int32),
            pltpu.VMEM((2, J), jnp.int32),
            pltpu.VMEM((2, J), jnp.int32),
            pltpu.VMEM((2, J), jnp.int32),
            pltpu.VMEM((J,), jnp.int32),
            pltpu.VMEM((2, J), jnp.float32),
            pltpu.VMEM((2, J), jnp.float32),
            pltpu.VMEM((2, J, 16), jnp.float32),
            pltpu.VMEM((2, J, 32), jnp.float32),
            pltpu.VMEM((16,), jnp.float32),
            pltpu.VMEM_SHARED((N_PAD, 32), jnp.float32),
            pltpu.SemaphoreType.DMA,
            pltpu.SemaphoreType.DMA,
            pltpu.SemaphoreType.DMA,
            pltpu.SemaphoreType.DMA,
        ],
    )
    def k(src_hbm, dst_hbm, htab_hbm, asrc_hbm, adst_hbm, zeros_hbm,
          consts_hbm, out_hbm, src_v, dst_v, hidx_v, didx_v, sidx_v,
          sa_v, la_v, rows_v, msg_v, consts_v, acc_sh,
          isem0, isem1, gsem0, gsem1):
        cid = lax.axis_index("c")
        sid = lax.axis_index("s")
        isems = (isem0, isem1)
        gsems = (gsem0, gsem1)

        pltpu.sync_copy(consts_hbm.at[cid], consts_v)
        pltpu.sync_copy(zeros_hbm.at[pl.ds(sid * rpt, rpt)],
                        acc_sh.at[pl.ds(sid * rpt, rpt)])
        plsc.subcore_barrier()

        cv = consts_v[...]
        hoff = cid * N_NODES

        def base(t):
            return (sid + t * NS) * J

        def idx_pairs(t, b):
            return ((src_hbm.at[pl.ds(base(t), J)], src_v.at[b]),
                    (dst_hbm.at[pl.ds(base(t), J)], dst_v.at[b]))

        def math(b):
            for g in range(G):
                sl = pl.ds(g * 16, 16)
                hidx_v[b, sl] = src_v[b, sl] + hoff
                didx_v[b, sl] = dst_v[b, sl] + hoff

        def gat_pairs(b):
            return ((asrc_hbm.at[hidx_v.at[b]], sa_v.at[b]),
                    (adst_hbm.at[didx_v.at[b]], la_v.at[b]),
                    (htab_hbm.at[hidx_v.at[b]], rows_v.at[b]))

        for s_, d_ in idx_pairs(0, 0):
            pltpu.sync_copy(s_, d_)
        math(0)
        for s_, d_ in gat_pairs(0):
            pltpu.async_copy(s_, d_, gsems[0])
        for s_, d_ in idx_pairs(1, 1):
            pltpu.async_copy(s_, d_, isems[1])

        def step(t, b):
            nb = 1 - b
            for s_, d_ in idx_pairs(t + 1, nb):
                pltpu.make_async_copy(s_, d_, isems[nb]).wait()
            math(nb)
            for s_, d_ in gat_pairs(nb):
                pltpu.async_copy(s_, d_, gsems[nb])
            for s_, d_ in gat_pairs(b):
                pltpu.make_async_copy(s_, d_, gsems[b]).wait()
            for g in range(G):
                sl = pl.ds(g * 16, 16)
                sidx_v[sl] = dst_v[b, sl]
            for s_, d_ in idx_pairs(t + 2, b):
                pltpu.async_copy(s_, d_, isems[b])
            for g in range(G):
                sl = pl.ds(g * 16, 16)
                sa = sa_v[b, sl]
                la = la_v[b, sl]
                pv = jnp.exp(_leaky(sa + la) - _leaky(cv + la))
                for i in range(16):
                    j = g * 16 + i
                    pb = jnp.broadcast_to(pv[i], (16,))
                    msg_v[b, j, pl.ds(0, 16)] = rows_v[b, j, pl.ds(0, 16)] * pb
                    msg_v[b, j, pl.ds(16, 16)] = pb
            pltpu.sync_copy(msg_v.at[b], acc_sh.at[sidx_v], add=True)

        def body(tt, carry):
            step(2 * tt, 0)
            step(2 * tt + 1, 1)
            return carry

        lax.fori_loop(0, TRIP2 // 2, body, 0)

        for s_, d_ in gat_pairs(0):
            pltpu.make_async_copy(s_, d_, gsems[0]).wait()
        for s_, d_ in idx_pairs(TRIP2 + 1, 1):
            pltpu.make_async_copy(s_, d_, isems[1]).wait()

        plsc.subcore_barrier()
        pltpu.sync_copy(acc_sh.at[pl.ds(sid * rpt, rpt)],
                        out_hbm.at[cid, pl.ds(sid * rpt, rpt)])

    return k


def _make_edge1_kernel():
    rpt = N_PAD // NS
    mesh = plsc.VectorSubcoreMesh(core_axis_name="c", subcore_axis_name="s",
                                  num_cores=NC, num_subcores=NS)

    @functools.partial(
        pl.kernel,
        mesh=mesh,
        compiler_params=pltpu.CompilerParams(use_tc_tiling_on_sc=False),
        out_type=jax.ShapeDtypeStruct((NC, N_PAD, 32), jnp.float32),
        scratch_types=[
            pltpu.VMEM((2, J), jnp.int32),
            pltpu.VMEM((2, J), jnp.int32),
            pltpu.VMEM((J,), jnp.int32),
            pltpu.VMEM((2, J), jnp.float32),
            pltpu.VMEM((2, J), jnp.float32),
            pltpu.VMEM((2, J, 16), jnp.float32),
            pltpu.VMEM((2, J, 32), jnp.float32),
            pltpu.VMEM((16,), jnp.float32),
            pltpu.VMEM_SHARED((N_PAD, 32), jnp.float32),
            pltpu.SemaphoreType.DMA,
            pltpu.SemaphoreType.DMA,
            pltpu.SemaphoreType.DMA,
            pltpu.SemaphoreType.DMA,
        ],
    )
    def k(src_hbm, dst_hbm, htab_hbm, asrc_hbm, adst_hbm, zeros_hbm,
          consts_hbm, out_hbm, src_v, dst_v, sidx_v, sa_v, la_v,
          rows_v, msg_v, consts_v, acc_sh, isem0, isem1, gsem0, gsem1):
        cid = lax.axis_index("c")
        sid = lax.axis_index("s")
        wid = sid * NC + cid
        isems = (isem0, isem1)
        gsems = (gsem0, gsem1)

        pltpu.sync_copy(consts_hbm, consts_v)
        pltpu.sync_copy(zeros_hbm.at[pl.ds(sid * rpt, rpt)],
                        acc_sh.at[pl.ds(sid * rpt, rpt)])
        plsc.subcore_barrier()

        cv = consts_v[...]

        def base(t):
            return (wid + t * NW) * J

        def idx_pairs(t, b):
            return ((src_hbm.at[pl.ds(base(t), J)], src_v.at[b]),
                    (dst_hbm.at[pl.ds(base(t), J)], dst_v.at[b]))

        def gat_pairs(b):
            return ((asrc_hbm.at[src_v.at[b]], sa_v.at[b]),
                    (adst_hbm.at[dst_v.at[b]], la_v.at[b]),
                    (htab_hbm.at[src_v.at[b]], rows_v.at[b]))

        for s_, d_ in idx_pairs(0, 0):
            pltpu.sync_copy(s_, d_)
        for s_, d_ in gat_pairs(0):
            pltpu.async_copy(s_, d_, gsems[0])
        for s_, d_ in idx_pairs(1, 1):
            pltpu.async_copy(s_, d_, isems[1])

        def step(t, b):
            nb = 1 - b
            for s_, d_ in idx_pairs(t + 1, nb):
                pltpu.make_async_copy(s_, d_, isems[nb]).wait()
            for s_, d_ in gat_pairs(nb):
                pltpu.async_copy(s_, d_, gsems[nb])
            for s_, d_ in gat_pairs(b):
                pltpu.make_async_copy(s_, d_, gsems[b]).wait()
            for g in range(G):
                sl = pl.ds(g * 16, 16)
                sidx_v[sl] = dst_v[b, sl]
            for s_, d_ in idx_pairs(t + 2, b):
                pltpu.async_copy(s_, d_, isems[b])
            for g in range(G):
                sl = pl.ds(g * 16, 16)
                sa = sa_v[b, sl]
                la = la_v[b, sl]
                pv = jnp.exp(_leaky(sa + la) - _leaky(cv + la))
                for i in range(16):
                    j = g * 16 + i
                    pb = jnp.broadcast_to(pv[i], (16,))
                    msg_v[b, j, pl.ds(0, 16)] = rows_v[b, j, pl.ds(0, 16)] * pb
                    msg_v[b, j, pl.ds(16, 16)] = pb
            pltpu.sync_copy(msg_v.at[b], acc_sh.at[sidx_v], add=True)

        def body(tt, carry):
            step(2 * tt, 0)
            step(2 * tt + 1, 1)
            return carry

        lax.fori_loop(0, TRIP1 // 2, body, 0)

        for s_, d_ in gat_pairs(0):
            pltpu.make_async_copy(s_, d_, gsems[0]).wait()
        for s_, d_ in idx_pairs(TRIP1 + 1, 1):
            pltpu.make_async_copy(s_, d_, isems[1]).wait()

        plsc.subcore_barrier()
        pltpu.sync_copy(acc_sh.at[pl.ds(sid * rpt, rpt)],
                        out_hbm.at[cid, pl.ds(sid * rpt, rpt)])

    return k


@functools.lru_cache(maxsize=None)
def _edge_kernel(heads):
    return _make_edge2_kernel() if heads == 2 else _make_edge1_kernel()


def _att_mat(att):
    heads, chan = att.shape
    cols = []
    for h in range(heads):
        parts = [jnp.zeros((chan,), jnp.float32)] * heads
        parts[h] = att[h]
        cols.append(jnp.concatenate(parts))
    return jnp.stack(cols, axis=1)


def kernel(x, edge_index, W1, as1, ad1, b1, W2, as2, ad2, b2, W3, as3, ad3,
           b3, Wout, bout):
    ei = edge_index.astype(jnp.int32)
    src = jnp.concatenate(
        [ei[0], jnp.zeros((EP - N_EDGES,), jnp.int32)])
    dst = jnp.concatenate(
        [ei[1], jnp.full((EP - N_EDGES,), N_NODES, jnp.int32)])
    zeros32 = jnp.zeros((N_PAD, 32), jnp.float32)

    asm1, adm1 = _att_mat(as1), _att_mat(ad1)
    asm2, adm2 = _att_mat(as2), _att_mat(ad2)
    asm3, adm3 = _att_mat(as3), _att_mat(ad3)

    def _tables2(tab, adst, m):
        htab = tab[:, 0:32].reshape(N_NODES, 2, 16).transpose(1, 0, 2)
        htab = htab.reshape(2 * N_NODES, 16)
        asrcT = jnp.pad(tab[:, 32:34].T.reshape(2 * N_NODES), (0, 16))
        adstT = jnp.pad(adst[:, 0:2].T.reshape(2 * N_NODES), (0, 16))
        cm = jnp.broadcast_to(m[0][:, None], (2, 16))
        return htab, asrcT, adstT, cm

    tab1, adst1, m1 = _t1(x, W1, asm1, adm1)
    ht1, asT1, adT1, cm1 = _tables2(tab1, adst1, m1)
    acc1 = _edge_kernel(2)(src, dst, ht1, asT1, adT1, zeros32, cm1)

    t2 = _make_merge(2, 16)
    tab2, adst2, m2 = t2(acc1, tab1, adst1, m1, b1[None, :], W2, asm2, adm2)
    ht2, asT2, adT2, cm2 = _tables2(tab2, adst2, m2)
    acc2 = _edge_kernel(2)(src, dst, ht2, asT2, adT2, zeros32, cm2)

    t3 = _make_merge(1, 8)
    tab3, adst3, m3 = t3(acc2, tab2, adst2, m2, b2[None, :], W3, asm3, adm3)
    asT3 = jnp.pad(tab3[:, 8], (0, 16))
    adT3 = jnp.pad(adst3[:, 8], (0, 16))
    cm3 = jnp.broadcast_to(m3[0, 0], (16,))
    acc3 = _edge_kernel(1)(src, dst, tab3, asT3, adT3, zeros32, cm3)

    return _t4(acc3, tab3, adst3, m3, b3[None, :], Wout, bout[None, :])
